# Initial kernel scaffold; baseline (speedup 1.0000x reference)
import jax
import jax.numpy as jnp
from jax.experimental import pallas as pl


def kernel(enc_input, ranking, pos_table):
    raise NotImplementedError("write your pallas kernel here")


# speedup vs baseline: 2.1737x; 2.1737x over previous
import functools

import jax
import jax.numpy as jnp
from jax import lax
from jax.experimental import pallas as pl
from jax.experimental.pallas import tpu as pltpu
from jax.experimental.pallas import tpu_sc as plsc

_D = 64
_NPOS = 200
_LANES = 16
_NW = 32
_CHUNK = 512


def _pe_kernel(enc_hbm, idx_hbm, tab_hbm, out_hbm,
               tab_v, idx_v, buf_v,
               sem_tab, sem_in0, sem_in1, sem_out0, sem_out1):
    rows = enc_hbm.shape[0]
    rows_per_w = rows // _NW
    n_chunks = rows_per_w // _CHUNK

    wid = lax.axis_index("s") * 2 + lax.axis_index("c")
    row0 = wid * rows_per_w

    sem_in = (sem_in0, sem_in1)
    sem_out = (sem_out0, sem_out1)

    pltpu.make_async_copy(tab_hbm, tab_v, sem_tab).start()

    def in_copy(g, s):
        base = row0 + g * _CHUNK
        return pltpu.make_async_copy(
            enc_hbm.at[pl.ds(base, _CHUNK), :], buf_v.at[s], sem_in[s])

    def idx_copy(g, s):
        base = row0 + g * _CHUNK
        return pltpu.make_async_copy(
            idx_hbm.at[pl.ds(base, _CHUNK)], idx_v.at[s], sem_in[s])

    def out_copy(g, s):
        base = row0 + g * _CHUNK
        return pltpu.make_async_copy(
            buf_v.at[s], out_hbm.at[pl.ds(base, _CHUNK), :], sem_out[s])

    def start_in(g, s):
        in_copy(g, s).start()
        idx_copy(g, s).start()

    def wait_in(g, s):
        in_copy(g, s).wait()
        idx_copy(g, s).wait()

    start_in(0, 0)
    start_in(1, 1)
    pltpu.make_async_copy(tab_hbm, tab_v, sem_tab).wait()

    def do_chunk(g, s):
        wait_in(g, s)

        def group_body(gr, carry):
            r0 = gr * _LANES
            iv = idx_v[s, pl.ds(r0, _LANES)]
            for k in range(_LANES):
                i = iv[k]
                for j in range(_D // _LANES):
                    t = tab_v[i, pl.ds(j * _LANES, _LANES)]
                    plsc.addupdate(
                        buf_v.at[s, r0 + k, pl.ds(j * _LANES, _LANES)], t)
            return carry

        lax.fori_loop(0, _CHUNK // _LANES, group_body, 0)

        out_copy(g, s).start()
        out_copy(g, s).wait()

        @pl.when(g + 2 < n_chunks)
        def _():
            start_in(g + 2, s)

    def pair_body(g2, carry):
        do_chunk(2 * g2, 0)
        do_chunk(2 * g2 + 1, 1)
        return carry

    lax.fori_loop(0, n_chunks // 2, pair_body, 0)


def kernel(enc_input, ranking, pos_table):
    b, l, d = enc_input.shape
    rows = b * l
    enc_flat = enc_input.reshape(rows, d)
    idx_flat = ranking.reshape(rows).astype(jnp.int32)
    tab = pos_table.reshape(_NPOS, d)

    mesh = plsc.VectorSubcoreMesh(core_axis_name="c", subcore_axis_name="s")
    run = pl.kernel(
        _pe_kernel,
        compiler_params=pltpu.CompilerParams(use_tc_tiling_on_sc=False),
        out_type=jax.ShapeDtypeStruct((rows, d), jnp.float32),
        mesh=mesh,
        scratch_types=[
            pltpu.VMEM((_NPOS, d), jnp.float32),
            pltpu.VMEM((2, _CHUNK), jnp.int32),
            pltpu.VMEM((2, _CHUNK, d), jnp.float32),
            pltpu.SemaphoreType.DMA,
            pltpu.SemaphoreType.DMA,
            pltpu.SemaphoreType.DMA,
            pltpu.SemaphoreType.DMA,
            pltpu.SemaphoreType.DMA,
        ],
    )
    out = run(enc_flat, idx_flat, tab)
    return out.reshape(1, b, l, d)
